# baseline (speedup 1.0000x reference)
import functools

import jax
import jax.numpy as jnp
from jax import lax
from jax.experimental import pallas as pl
from jax.experimental.pallas import tpu as pltpu
from jax.experimental.pallas import tpu_sc as plsc

HIDDEN = 64
SCALE = 8.0

NC = 2
NS = 16
NW = NC * NS

C = 128
LANES = 16
ROWS_PER_IT = 8


def _make_emb_kernel(B):
    assert B % NW == 0
    bpw = B // NW
    assert bpw % C == 0
    nchunk = bpw // C
    assert nchunk % 2 == 0

    mesh = plsc.VectorSubcoreMesh(core_axis_name="c", subcore_axis_name="s")

    @functools.partial(
        pl.kernel,
        mesh=mesh,
        out_type=jax.ShapeDtypeStruct((B, HIDDEN), jnp.float32),
        compiler_params=pltpu.CompilerParams(use_tc_tiling_on_sc=False),
        scratch_types=[
            pltpu.VMEM((bpw,), jnp.int32),
            pltpu.VMEM((2, C, HIDDEN), jnp.float32),
            pltpu.SemaphoreType.DMA,
            pltpu.SemaphoreType.DMA,
        ],
    )
    def emb(idx_hbm, tab_hbm, out_hbm, idx_v, rows_v, sem0, sem1):
        wid = lax.axis_index("s") * NC + lax.axis_index("c")
        base = wid * bpw
        sems = (sem0, sem1)

        pltpu.sync_copy(idx_hbm.at[pl.ds(base, bpw)], idx_v)

        def start(g, slot):
            pltpu.async_copy(
                tab_hbm.at[idx_v.at[pl.ds(g * C, C)]],
                rows_v.at[slot],
                sems[slot],
            )

        def wait(g, slot):
            pltpu.make_async_copy(
                tab_hbm.at[idx_v.at[pl.ds(g * C, C)]],
                rows_v.at[slot],
                sems[slot],
            ).wait()

        def scale_rows(slot):
            def body(j, carry):
                r0 = j * ROWS_PER_IT
                for jj in range(ROWS_PER_IT):
                    for k in range(HIDDEN // LANES):
                        v = rows_v[slot, r0 + jj, pl.ds(k * LANES, LANES)]
                        rows_v[slot, r0 + jj, pl.ds(k * LANES, LANES)] = v * SCALE
                return carry
            lax.fori_loop(0, C // ROWS_PER_IT, body, 0)

        def finish(g, slot):
            wait(g, slot)
            scale_rows(slot)
            pltpu.sync_copy(rows_v.at[slot], out_hbm.at[pl.ds(base + g * C, C)])

        start(0, 0)

        def pair(p, carry):
            g0 = 2 * p
            start(g0 + 1, 1)
            finish(g0, 0)
            start(g0 + 2, 0)
            finish(g0 + 1, 1)
            return carry

        lax.fori_loop(0, nchunk // 2 - 1, pair, 0)

        g0 = nchunk - 2
        start(g0 + 1, 1)
        finish(g0, 0)
        finish(g0 + 1, 1)

    return emb


def kernel(x, weights):
    b, s = x.shape
    xf = x.reshape(-1).astype(jnp.int32)
    out = _make_emb_kernel(xf.shape[0])(xf, weights)
    return out.reshape(b, s, HIDDEN)
